# baseline (speedup 1.0000x reference)
import functools

import jax
import jax.numpy as jnp
from jax import lax
from jax.experimental import pallas as pl
from jax.experimental.pallas import tpu as pltpu

B, S, DIM, NE, TOPK, RANK = 2, 2048, 1024, 8, 2, 32
T = B * S
H = 4 * DIM
CAP = max(1, int(T * 1.25 / NE))
TB = 1024
HB = 512
NTB = T // TB
NHB = H // HB
NEXP = NE + 1


def _cumsum0(a):
    n = a.shape[0]
    sh = 1
    while sh < n:
        z = jnp.zeros((sh, a.shape[1]), a.dtype)
        a = a + jnp.concatenate([z, a[:-sh, :]], axis=0)
        sh *= 2
    return a


def _router_kernel(x_ref, rw_ref, probs_ref, idx_ref, w_ref, zl_ref, cnt_ref):
    tb = pl.program_id(0)
    x = x_ref[...]
    rw = rw_ref[...]
    logits = lax.dot_general(x, rw, (((1,), (0,)), ((), ())),
                             preferred_element_type=jnp.float32)
    m = jnp.max(logits, axis=1, keepdims=True)
    ex = jnp.exp(logits - m)
    s = jnp.sum(ex, axis=1, keepdims=True)
    full = ex / s
    lse = m + jnp.log(s)
    zsum = jnp.sum(lse * lse)
    prev_z = jnp.where(tb == 0, 0.0, zl_ref[0, 0])
    zl_ref[0, 0] = prev_z + zsum

    lane = lax.broadcasted_iota(jnp.int32, (TB, NE), 1)
    v1 = jnp.max(full, axis=1, keepdims=True)
    i1 = jnp.min(jnp.where(full == v1, lane, NE), axis=1, keepdims=True)
    masked = jnp.where(lane == i1, -1.0, full)
    v2 = jnp.max(masked, axis=1, keepdims=True)
    i2 = jnp.min(jnp.where(masked == v2, lane, NE), axis=1, keepdims=True)
    denom = v1 + v2
    p1 = v1 / denom
    p2 = v2 / denom
    probs_ref[...] = jnp.concatenate([p1, p2], axis=1)
    idx_ref[...] = jnp.concatenate([i1, i2], axis=1)

    oh1 = (lane == i1).astype(jnp.float32)
    oh2 = (lane == i2).astype(jnp.float32)
    cs1 = _cumsum0(oh1)
    cs2 = _cumsum0(oh2)
    prev = jnp.where(tb == 0, 0.0, cnt_ref[...])
    base1 = prev[0:1, :]
    base2 = prev[1:2, :]
    rank1 = jnp.sum(oh1 * (cs1 + base1), axis=1, keepdims=True)
    rank2 = jnp.sum(oh2 * (cs2 + base2), axis=1, keepdims=True)
    keep1 = (rank1 <= CAP).astype(jnp.float32)
    keep2 = (rank2 <= CAP).astype(jnp.float32)
    w_ref[...] = oh1 * (p1 * keep1) + oh2 * (p2 * keep2)
    cnt_ref[...] = prev + jnp.concatenate(
        [jnp.sum(oh1, axis=0, keepdims=True),
         jnp.sum(oh2, axis=0, keepdims=True)], axis=0)


def _router_call(xf, router_w):
    return pl.pallas_call(
        _router_kernel,
        grid=(NTB,),
        in_specs=[
            pl.BlockSpec((TB, DIM), lambda tb: (tb, 0)),
            pl.BlockSpec((DIM, NE), lambda tb: (0, 0)),
        ],
        out_specs=[
            pl.BlockSpec((TB, TOPK), lambda tb: (tb, 0)),
            pl.BlockSpec((TB, TOPK), lambda tb: (tb, 0)),
            pl.BlockSpec((TB, NE), lambda tb: (tb, 0)),
            pl.BlockSpec(memory_space=pltpu.SMEM),
        ],
        out_shape=[
            jax.ShapeDtypeStruct((T, TOPK), jnp.float32),
            jax.ShapeDtypeStruct((T, TOPK), jnp.int32),
            jax.ShapeDtypeStruct((T, NE), jnp.float32),
            jax.ShapeDtypeStruct((1, 1), jnp.float32),
        ],
        scratch_shapes=[pltpu.VMEM((2, NE), jnp.float32)],
    )(xf, router_w)


def _gelu(a):
    return 0.5 * a * (1.0 + lax.erf(a * 0.7071067811865476))


def _expert_kernel(x_ref, w1_ref, w2_ref, b1_ref, b2_ref, dw_ref, uw_ref,
                   wt_ref, out_ref, acc_ref):
    e = pl.program_id(1)
    hb = pl.program_id(2)
    x = x_ref[...]
    w1 = w1_ref[0]
    b1 = b1_ref[pl.ds(e, 1), pl.ds(hb * HB, HB)]
    a = lax.dot_general(x, w1, (((1,), (1,)), ((), ())),
                        preferred_element_type=jnp.float32)
    a = _gelu(a + b1)
    w2 = w2_ref[0]
    contrib = lax.dot_general(a.astype(jnp.bfloat16), w2,
                              (((1,), (1,)), ((), ())),
                              preferred_element_type=jnp.float32)

    dw = dw_ref[pl.program_id(1)]
    uw = uw_ref[pl.program_id(1)]
    ad1 = lax.dot_general(x, dw, (((1,), (1,)), ((), ())),
                          preferred_element_type=jnp.float32)
    ad = lax.dot_general(_gelu(ad1).astype(jnp.bfloat16), uw,
                         (((1,), (1,)), ((), ())),
                         preferred_element_type=jnp.float32)
    b2 = b2_ref[pl.ds(e, 1), :]
    init = ad + b2
    prev = jnp.where(hb == 0, init, acc_ref[...])
    acc = prev + contrib
    acc_ref[...] = acc

    @pl.when(hb == NHB - 1)
    def _():
        wt = wt_ref[...]
        lane = lax.broadcasted_iota(jnp.int32, (TB, NEXP), 1)
        col = jnp.sum(jnp.where(lane == e, wt, 0.0), axis=1, keepdims=True)
        out_prev = jnp.where(e == 0, 0.0, out_ref[...])
        out_ref[...] = out_prev + col * acc


def _expert_call(xbf, fc1s, fc2s, b1s, b2s, dws, uws, w9):
    return pl.pallas_call(
        _expert_kernel,
        grid=(NTB, NEXP, NHB),
        in_specs=[
            pl.BlockSpec((TB, DIM), lambda tb, e, hb: (tb, 0)),
            pl.BlockSpec((1, HB, DIM), lambda tb, e, hb: (e, hb, 0)),
            pl.BlockSpec((1, DIM, HB), lambda tb, e, hb: (e, 0, hb)),
            pl.BlockSpec((NEXP, H), lambda tb, e, hb: (0, 0)),
            pl.BlockSpec((NEXP, DIM), lambda tb, e, hb: (0, 0)),
            pl.BlockSpec((NEXP, RANK, DIM), lambda tb, e, hb: (0, 0, 0)),
            pl.BlockSpec((NEXP, DIM, RANK), lambda tb, e, hb: (0, 0, 0)),
            pl.BlockSpec((TB, NEXP), lambda tb, e, hb: (tb, 0)),
        ],
        out_specs=pl.BlockSpec((TB, DIM), lambda tb, e, hb: (tb, 0)),
        out_shape=jax.ShapeDtypeStruct((T, DIM), jnp.float32),
        scratch_shapes=[pltpu.VMEM((TB, DIM), jnp.float32)],
        compiler_params=pltpu.CompilerParams(
            dimension_semantics=("arbitrary", "arbitrary", "arbitrary"),
        ),
    )(xbf, fc1s, fc2s, b1s, b2s, dws, uws, w9)


def kernel(x, router_w, fc1_w, fc1_b, fc2_w, fc2_b, down_w, up_w,
           sh_fc1_w, sh_fc1_b, sh_fc2_w, sh_fc2_b, sh_down_w, sh_up_w):
    xf = x.reshape(T, DIM)
    probs, tidx, w8, zl = _router_call(xf, router_w)

    bf = jnp.bfloat16
    fc1s = jnp.concatenate([fc1_w, sh_fc1_w[None]], axis=0).astype(bf)
    fc2s = jnp.concatenate([fc2_w, sh_fc2_w[None]], axis=0).astype(bf)
    b1s = jnp.concatenate([fc1_b, sh_fc1_b[None]], axis=0)
    b2s = jnp.concatenate([fc2_b, sh_fc2_b[None]], axis=0)
    dws = jnp.concatenate([down_w, sh_down_w[None]], axis=0).astype(bf)
    uws = jnp.concatenate([up_w, sh_up_w[None]], axis=0).astype(bf)
    w9 = jnp.concatenate([w8, jnp.full((T, 1), 0.2, jnp.float32)], axis=1)

    out = _expert_call(xf.astype(bf), fc1s, fc2s, b1s, b2s, dws, uws, w9)

    z_loss = zl[0, 0] / T
    ema = jnp.full((NE,), 1.0 / NE, jnp.float32)
    return (out.reshape(B, S, DIM), probs.reshape(B, S, TOPK),
            tidx.reshape(B, S, TOPK), ema, z_loss)
